# Initial kernel scaffold; baseline (speedup 1.0000x reference)
import jax
import jax.numpy as jnp
from jax.experimental import pallas as pl


def kernel(node_feats, u_near, v_near, u_far, v_far, W1n, b1n, W2n, b2n, W1f, b1f, W2f, b2f, alpha, margin):
    raise NotImplementedError("write your pallas kernel here")


# speedup vs baseline: 33.3864x; 33.3864x over previous
import functools

import jax
import jax.numpy as jnp
from jax import lax
from jax.experimental import pallas as pl
from jax.experimental.pallas import tpu as pltpu
from jax.experimental.pallas import tpu_sc as plsc

_C = 10000
_E = 160000
_NBATCH = 4
_D = 256
_W = _NBATCH * _D
_NS = 16
_EP = _E // _NS
_WROWS = 100
_NWIN = _C // _WROWS
_WIN_TOT = _WROWS + 2
_CHUNK = 2000
_NCHUNK = _EP // _CHUNK
_VEC = 16
_BM = 400
_MB = _C // _BM


def _fill(ref, n, vec):
    def body(i, _):
        ref[pl.ds(i * _VEC, _VEC)] = vec
        return 0
    lax.fori_loop(0, n // _VEC, body, 0)


def _adj_body(edges, a_out, deg_out, ubuf, vbuf, idxbuf, ones_ck, zbuf,
              ones_ep, stagebuf, win_sh, deg_sh):
    g = lax.axis_index("c")
    s = lax.axis_index("s")

    zeros16f = jnp.zeros((_VEC,), jnp.float32)
    ones16f = jnp.ones((_VEC,), jnp.float32)

    _fill(zbuf, _C, zeros16f)
    _fill(ones_ep, _EP, ones16f)
    _fill(ones_ck, 2 * _CHUNK, ones16f)

    pltpu.sync_copy(edges.at[pl.ds(g * 2 * _E + s * _EP, _EP)], ubuf)
    pltpu.sync_copy(edges.at[pl.ds(g * 2 * _E + _E + s * _EP, _EP)], vbuf)

    @pl.when(s == 0)
    def _():
        pltpu.sync_copy(zbuf, deg_sh)
    plsc.subcore_barrier()
    pltpu.sync_copy(ones_ep, deg_sh.at[ubuf], add=True)
    pltpu.sync_copy(ones_ep, deg_sh.at[vbuf], add=True)
    plsc.subcore_barrier()

    @pl.when(s == 0)
    def _():
        pltpu.sync_copy(deg_sh, stagebuf)
        pltpu.sync_copy(stagebuf, deg_out.at[pl.ds(g * _C, _C)])

    def win_body(w, _):
        base = w * _WROWS

        def zero_rows(i, _):
            r = s + i * _NS

            @pl.when(r < _WROWS)
            def _():
                pltpu.sync_copy(zbuf, win_sh.at[pl.ds((r + 1) * _C, _C)])
            return 0
        lax.fori_loop(0, (_WROWS + _NS - 1) // _NS, zero_rows, 0)
        plsc.subcore_barrier()

        def chunk_body(c, _):
            def scan(i, _):
                off = c * _CHUNK + i * _VEC
                uv = ubuf[pl.ds(off, _VEC)]
                vv = vbuf[pl.ds(off, _VEC)]
                relv = jnp.clip(vv - base, -1, _WROWS)
                idxbuf[pl.ds(i * _VEC, _VEC)] = (relv + 1) * _C + uv
                relu = jnp.clip(uv - base, -1, _WROWS)
                idxbuf[pl.ds(_CHUNK + i * _VEC, _VEC)] = (relu + 1) * _C + vv
                return 0
            lax.fori_loop(0, _CHUNK // _VEC, scan, 0)
            pltpu.sync_copy(ones_ck, win_sh.at[idxbuf], add=True)
            return 0
        lax.fori_loop(0, _NCHUNK, chunk_body, 0)
        plsc.subcore_barrier()

        def writeback(i, _):
            r = s + i * _NS

            @pl.when(r < _WROWS)
            def _():
                pltpu.sync_copy(win_sh.at[pl.ds((r + 1) * _C, _C)], stagebuf)
                pltpu.sync_copy(stagebuf,
                                a_out.at[pl.ds((g * _C + base + r) * _C, _C)])
            return 0
        lax.fori_loop(0, (_WROWS + _NS - 1) // _NS, writeback, 0)
        plsc.subcore_barrier()
        return 0

    lax.fori_loop(0, _NWIN, win_body, 0)


def _build_adj(edges):
    mesh = plsc.VectorSubcoreMesh(core_axis_name="c", subcore_axis_name="s")
    f = pl.kernel(
        _adj_body,
        out_type=(jax.ShapeDtypeStruct((2 * _C * _C,), jnp.float32),
                  jax.ShapeDtypeStruct((2 * _C,), jnp.float32)),
        mesh=mesh,
        scratch_types=[
            pltpu.VMEM((_EP,), jnp.int32),
            pltpu.VMEM((_EP,), jnp.int32),
            pltpu.VMEM((2 * _CHUNK,), jnp.int32),
            pltpu.VMEM((2 * _CHUNK,), jnp.float32),
            pltpu.VMEM((_C,), jnp.float32),
            pltpu.VMEM((_EP,), jnp.float32),
            pltpu.VMEM((_C,), jnp.float32),
            pltpu.VMEM_SHARED((_WIN_TOT * _C,), jnp.float32),
            pltpu.VMEM_SHARED((_C,), jnp.float32),
        ],
    )
    return f(edges)


def _loss_gather_body(edges, gmat, nx, out, ubuf, vbuf, idxbuf, valbuf):
    g = lax.axis_index("c")
    s = lax.axis_index("s")
    pltpu.sync_copy(edges.at[pl.ds(g * 2 * _E + s * _EP, _EP)], ubuf)
    pltpu.sync_copy(edges.at[pl.ds(g * 2 * _E + _E + s * _EP, _EP)], vbuf)

    def body(i, _):
        uv = ubuf[pl.ds(i * _VEC, _VEC)]
        vv = vbuf[pl.ds(i * _VEC, _VEC)]
        idxbuf[pl.ds(i * _VEC, _VEC)] = uv * _C + vv
        return 0
    lax.fori_loop(0, _EP // _VEC, body, 0)

    base = g * 3 * _E + s * _EP
    pltpu.sync_copy(gmat.at[idxbuf], valbuf)
    pltpu.sync_copy(valbuf, out.at[pl.ds(base, _EP)])
    pltpu.sync_copy(nx.at[ubuf], valbuf)
    pltpu.sync_copy(valbuf, out.at[pl.ds(base + _E, _EP)])
    pltpu.sync_copy(nx.at[vbuf], valbuf)
    pltpu.sync_copy(valbuf, out.at[pl.ds(base + 2 * _E, _EP)])


def _loss_gather(edges, gmat_flat, nx_flat):
    mesh = plsc.VectorSubcoreMesh(core_axis_name="c", subcore_axis_name="s")
    f = pl.kernel(
        _loss_gather_body,
        out_type=jax.ShapeDtypeStruct((2 * 3 * _E,), jnp.float32),
        mesh=mesh,
        scratch_types=[
            pltpu.VMEM((_EP,), jnp.int32),
            pltpu.VMEM((_EP,), jnp.int32),
            pltpu.VMEM((_EP,), jnp.int32),
            pltpu.VMEM((_EP,), jnp.float32),
        ],
    )
    return f(edges, gmat_flat, nx_flat)


def _cast_body(a_ref, o_ref):
    o_ref[...] = a_ref[...].astype(jnp.bfloat16)


def _cast_bf16(a):
    return pl.pallas_call(
        _cast_body,
        grid=(2, _MB),
        in_specs=[pl.BlockSpec((1, _BM, _C), lambda g, i: (g, i, 0))],
        out_specs=pl.BlockSpec((1, _BM, _C), lambda g, i: (g, i, 0)),
        out_shape=jax.ShapeDtypeStruct((2, _C, _C), jnp.bfloat16),
    )(a)


def _prep_body(x_ref, degn_ref, degf_ref, y1_ref, nn_ref, nf_ref, nx_ref):
    x = x_ref[...]
    nn = lax.rsqrt(jnp.maximum(degn_ref[...], 1.0))
    nf = lax.rsqrt(jnp.maximum(degf_ref[...], 1.0))
    nn_ref[...] = nn
    nf_ref[...] = nf
    y1_ref[...] = (x * nn).astype(jnp.bfloat16)
    x0 = x[:, :_D]
    nx_ref[...] = jnp.sum(x0 * x0, axis=1, keepdims=True)


def _prep(x, degn, degf):
    return pl.pallas_call(
        _prep_body,
        grid=(_MB,),
        in_specs=[
            pl.BlockSpec((_BM, _W), lambda i: (i, 0)),
            pl.BlockSpec((_BM, 1), lambda i: (i, 0)),
            pl.BlockSpec((_BM, 1), lambda i: (i, 0)),
        ],
        out_specs=[
            pl.BlockSpec((_BM, _W), lambda i: (i, 0)),
            pl.BlockSpec((_BM, 1), lambda i: (i, 0)),
            pl.BlockSpec((_BM, 1), lambda i: (i, 0)),
            pl.BlockSpec((_BM, 1), lambda i: (i, 0)),
        ],
        out_shape=[
            jax.ShapeDtypeStruct((_C, _W), jnp.bfloat16),
            jax.ShapeDtypeStruct((_C, 1), jnp.float32),
            jax.ShapeDtypeStruct((_C, 1), jnp.float32),
            jax.ShapeDtypeStruct((_C, 1), jnp.float32),
        ],
    )(x, degn, degf)


def _layer_body(relu, final, a_ref, y_ref, no_ref, nn_ref, w_ref, b_ref,
                o_ref):
    a = a_ref[...]
    y = y_ref[...]
    acc = jnp.dot(a, y, preferred_element_type=jnp.float32)
    z = (acc * no_ref[...]).astype(jnp.bfloat16)
    wmat = w_ref[...].astype(jnp.bfloat16)
    b = b_ref[...]
    cols = []
    for j in range(_NBATCH):
        h = jnp.dot(z[:, j * _D:(j + 1) * _D], wmat,
                    preferred_element_type=jnp.float32) + b
        if relu:
            h = jnp.maximum(h, 0.0)
        cols.append(h)
    h = jnp.concatenate(cols, axis=1)
    if final:
        o_ref[...] = h
    else:
        o_ref[...] = (h * nn_ref[...]).astype(jnp.bfloat16)


def _layer(a, y, norm_out, norm_next, wmat, b, relu, final):
    odtype = jnp.float32 if final else jnp.bfloat16
    return pl.pallas_call(
        functools.partial(_layer_body, relu, final),
        grid=(_MB,),
        in_specs=[
            pl.BlockSpec((_BM, _C), lambda i: (i, 0)),
            pl.BlockSpec((_C, _W), lambda i: (0, 0)),
            pl.BlockSpec((_BM, 1), lambda i: (i, 0)),
            pl.BlockSpec((_BM, 1), lambda i: (i, 0)),
            pl.BlockSpec((_D, _D), lambda i: (0, 0)),
            pl.BlockSpec((1, _D), lambda i: (0, 0)),
        ],
        out_specs=pl.BlockSpec((_BM, _W), lambda i: (i, 0)),
        out_shape=jax.ShapeDtypeStruct((_C, _W), odtype),
    )(a, y, norm_out, norm_next, wmat, b.reshape(1, _D))


_BG = 200


def _gram_body(xa_ref, xb_ref, g_ref):
    xa = xa_ref[...].astype(jnp.bfloat16)
    xb = xb_ref[...].astype(jnp.bfloat16)
    g_ref[...] = lax.dot_general(xa, xb, (((1,), (1,)), ((), ())),
                                 preferred_element_type=jnp.float32)


def _gram(x0):
    return pl.pallas_call(
        _gram_body,
        grid=(_C // _BG,),
        in_specs=[
            pl.BlockSpec((_BG, _D), lambda i: (i, 0)),
            pl.BlockSpec((_C, _D), lambda i: (0, 0)),
        ],
        out_specs=pl.BlockSpec((_BG, _C), lambda i: (i, 0)),
        out_shape=jax.ShapeDtypeStruct((_C, _C), jnp.float32),
    )(x0, x0)


_ER, _EC = 1250, 128


def _loss_body(vn_ref, vf_ref, uf_ref, vfi_ref, al_ref, mg_ref, o_ref):
    margin = mg_ref[0, 0]
    gn = vn_ref[0]
    dt2n = jnp.maximum(vn_ref[1] + vn_ref[2] - 2.0 * gn, 0.0)
    touches = jnp.sum(dt2n) / _E

    eq = uf_ref[...] == vfi_ref[...]
    dt2f = jnp.where(eq, 0.0, jnp.maximum(vf_ref[1] + vf_ref[2]
                                          - 2.0 * vf_ref[0], 0.0))
    dd = jnp.sqrt(dt2f)
    hinge = jnp.maximum(margin - dd, 0.0)
    disjoint = jnp.sum(hinge * hinge) / _E

    a = jax.nn.sigmoid(al_ref[0, 0])
    o_ref[...] = jnp.reshape(a * touches + (1.0 - a) * disjoint, (1, 1))


def _loss(vn, vf, uf, vfi, alpha, margin):
    return pl.pallas_call(
        _loss_body,
        grid=(1,),
        in_specs=[
            pl.BlockSpec((3, _ER, _EC), lambda i: (0, 0, 0)),
            pl.BlockSpec((3, _ER, _EC), lambda i: (0, 0, 0)),
            pl.BlockSpec((_ER, _EC), lambda i: (0, 0)),
            pl.BlockSpec((_ER, _EC), lambda i: (0, 0)),
            pl.BlockSpec((1, 1), lambda i: (0, 0)),
            pl.BlockSpec((1, 1), lambda i: (0, 0)),
        ],
        out_specs=pl.BlockSpec((1, 1), lambda i: (0, 0)),
        out_shape=jax.ShapeDtypeStruct((1, 1), jnp.float32),
    )(vn, vf, uf, vfi, alpha, margin)


def kernel(node_feats, u_near, v_near, u_far, v_far,
           W1n, b1n, W2n, b2n, W1f, b1f, W2f, b2f, alpha, margin):
    edges = jnp.concatenate([u_near, v_near, u_far, v_far])

    adj, deg = _build_adj(edges)
    adj = _cast_bf16(adj.reshape(2, _C, _C))
    deg = deg.reshape(2, _C)

    x = node_feats.transpose(1, 0, 2).reshape(_C, _W)
    y1, nn, nf, nx = _prep(x, deg[0].reshape(_C, 1), deg[1].reshape(_C, 1))

    y2 = _layer(adj[0], y1, nn, nn, W1n, b1n, relu=True, final=False)
    y3 = _layer(adj[0], y2, nn, nf, W2n, b2n, relu=False, final=False)
    y4 = _layer(adj[1], y3, nf, nf, W1f, b1f, relu=True, final=False)
    far = _layer(adj[1], y4, nf, nf, W2f, b2f, relu=False, final=True)
    new_node_feats = far.reshape(_C, _NBATCH, _D).transpose(1, 0, 2)

    gmat = _gram(node_feats[0])
    vals = _loss_gather(edges, gmat.reshape(-1), nx.reshape(-1))
    vals = vals.reshape(2, 3, _E)
    vn = vals[0].reshape(3, _ER, _EC)
    vf = vals[1].reshape(3, _ER, _EC)
    topo = _loss(vn, vf, u_far.reshape(_ER, _EC), v_far.reshape(_ER, _EC),
                 alpha.reshape(1, 1), margin.reshape(1, 1))[0, 0]

    return new_node_feats, topo


# speedup vs baseline: 38.0688x; 1.1402x over previous
import functools

import jax
import jax.numpy as jnp
from jax import lax
from jax.experimental import pallas as pl
from jax.experimental.pallas import tpu as pltpu
from jax.experimental.pallas import tpu_sc as plsc

_C = 10000
_E = 160000
_NBATCH = 4
_D = 256
_W = _NBATCH * _D
_NS = 16
_EP = _E // _NS
_WROWS = 100
_NWIN = _C // _WROWS
_WIN_TOT = _WROWS + 2
_CHUNK = 2000
_NCHUNK = _EP // _CHUNK
_VEC = 16
_BM = 400
_MB = _C // _BM


def _fill(ref, n, vec):
    def body(i, _):
        ref[pl.ds(i * _VEC, _VEC)] = vec
        return 0
    lax.fori_loop(0, n // _VEC, body, 0)


def _adj_body(edges, a_out, deg_out, ubuf, vbuf, idx0, idx1, ones_ck, zbuf,
              ones_ep, stage0, stage1, sems, win_sh):
    g = lax.axis_index("c")
    s = lax.axis_index("s")

    _fill(zbuf, _C, jnp.zeros((_VEC,), jnp.float32))
    _fill(ones_ep, _EP, jnp.ones((_VEC,), jnp.float32))
    _fill(ones_ck, 2 * _CHUNK, jnp.ones((_VEC,), jnp.float32))

    pltpu.sync_copy(edges.at[pl.ds(g * 2 * _E + s * _EP, _EP)], ubuf)
    pltpu.sync_copy(edges.at[pl.ds(g * 2 * _E + _E + s * _EP, _EP)], vbuf)

    @pl.when(s == 0)
    def _():
        pltpu.sync_copy(zbuf, win_sh.at[pl.ds(0, _C)])
    plsc.subcore_barrier()
    pltpu.sync_copy(ones_ep, win_sh.at[ubuf], add=True)
    pltpu.sync_copy(ones_ep, win_sh.at[vbuf], add=True)
    plsc.subcore_barrier()

    @pl.when(s == 0)
    def _():
        pltpu.sync_copy(win_sh.at[pl.ds(0, _C)], stage0)
        pltpu.sync_copy(stage0, deg_out.at[pl.ds(g * _C, _C)])
    plsc.subcore_barrier()

    idxbufs = (idx0, idx1)

    def win_body(w, _):
        base = w * _WROWS

        def zero_rows(i, _):
            r = s + i * _NS

            @pl.when(r < _WROWS)
            def _():
                pltpu.sync_copy(zbuf, win_sh.at[pl.ds((r + 1) * _C, _C)])
            return 0
        lax.fori_loop(0, (_WROWS + _NS - 1) // _NS, zero_rows, 0)
        plsc.subcore_barrier()

        def compute_chunk(c, buf):
            def scan(i, _):
                off = c * _CHUNK + i * _VEC
                uv = ubuf[pl.ds(off, _VEC)]
                vv = vbuf[pl.ds(off, _VEC)]
                relv = jnp.clip(vv - base, -1, _WROWS)
                buf[pl.ds(i * _VEC, _VEC)] = (relv + 1) * _C + uv
                relu = jnp.clip(uv - base, -1, _WROWS)
                buf[pl.ds(_CHUNK + i * _VEC, _VEC)] = (relu + 1) * _C + vv
                return 0
            lax.fori_loop(0, _CHUNK // _VEC, scan, 0)

        compute_chunk(0, idx0)
        for c in range(_NCHUNK):
            d = pltpu.async_copy(ones_ck, win_sh.at[idxbufs[c % 2]],
                                 sems[1 + c % 2], add=True)
            if c + 1 < _NCHUNK:
                compute_chunk(c + 1, idxbufs[(c + 1) % 2])
            d.wait()
        plsc.subcore_barrier()

        def writeback(i, _):
            r = s + i * _NS

            @pl.when(r < _WROWS)
            def _():
                pltpu.sync_copy(win_sh.at[pl.ds((r + 1) * _C, _C)], stage0)
                pltpu.sync_copy(stage0,
                                a_out.at[pl.ds((g * _C + base + r) * _C, _C)])
            return 0
        lax.fori_loop(0, (_WROWS + _NS - 1) // _NS, writeback, 0)
        plsc.subcore_barrier()
        return 0

    lax.fori_loop(0, _NWIN, win_body, 0)


def _build_adj(edges):
    mesh = plsc.VectorSubcoreMesh(core_axis_name="c", subcore_axis_name="s")
    f = pl.kernel(
        _adj_body,
        out_type=(jax.ShapeDtypeStruct((2 * _C * _C,), jnp.float32),
                  jax.ShapeDtypeStruct((2 * _C,), jnp.float32)),
        mesh=mesh,
        scratch_types=[
            pltpu.VMEM((_EP,), jnp.int32),
            pltpu.VMEM((_EP,), jnp.int32),
            pltpu.VMEM((2 * _CHUNK,), jnp.int32),
            pltpu.VMEM((2 * _CHUNK,), jnp.int32),
            pltpu.VMEM((2 * _CHUNK,), jnp.float32),
            pltpu.VMEM((_C,), jnp.float32),
            pltpu.VMEM((_EP,), jnp.float32),
            pltpu.VMEM((_C,), jnp.float32),
            pltpu.VMEM((_C,), jnp.float32),
            [pltpu.SemaphoreType.DMA] * 5,
            pltpu.VMEM_SHARED((_WIN_TOT * _C,), jnp.float32),
        ],
    )
    return f(edges)


def _loss_gather_body(edges, gmat, nx, out, ubuf, vbuf, idxbuf, valbuf):
    g = lax.axis_index("c")
    s = lax.axis_index("s")
    pltpu.sync_copy(edges.at[pl.ds(g * 2 * _E + s * _EP, _EP)], ubuf)
    pltpu.sync_copy(edges.at[pl.ds(g * 2 * _E + _E + s * _EP, _EP)], vbuf)

    def body(i, _):
        uv = ubuf[pl.ds(i * _VEC, _VEC)]
        vv = vbuf[pl.ds(i * _VEC, _VEC)]
        idxbuf[pl.ds(i * _VEC, _VEC)] = uv * _C + vv
        return 0
    lax.fori_loop(0, _EP // _VEC, body, 0)

    base = g * 3 * _E + s * _EP
    pltpu.sync_copy(gmat.at[idxbuf], valbuf)
    pltpu.sync_copy(valbuf, out.at[pl.ds(base, _EP)])
    pltpu.sync_copy(nx.at[ubuf], valbuf)
    pltpu.sync_copy(valbuf, out.at[pl.ds(base + _E, _EP)])
    pltpu.sync_copy(nx.at[vbuf], valbuf)
    pltpu.sync_copy(valbuf, out.at[pl.ds(base + 2 * _E, _EP)])


def _loss_gather(edges, gmat_flat, nx_flat):
    mesh = plsc.VectorSubcoreMesh(core_axis_name="c", subcore_axis_name="s")
    f = pl.kernel(
        _loss_gather_body,
        out_type=jax.ShapeDtypeStruct((2 * 3 * _E,), jnp.float32),
        mesh=mesh,
        scratch_types=[
            pltpu.VMEM((_EP,), jnp.int32),
            pltpu.VMEM((_EP,), jnp.int32),
            pltpu.VMEM((_EP,), jnp.int32),
            pltpu.VMEM((_EP,), jnp.float32),
        ],
    )
    return f(edges, gmat_flat, nx_flat)


def _cast_body(a_ref, o_ref):
    o_ref[...] = a_ref[...].astype(jnp.bfloat16)


def _cast_bf16(a):
    return pl.pallas_call(
        _cast_body,
        grid=(2, _MB),
        in_specs=[pl.BlockSpec((1, _BM, _C), lambda g, i: (g, i, 0))],
        out_specs=pl.BlockSpec((1, _BM, _C), lambda g, i: (g, i, 0)),
        out_shape=jax.ShapeDtypeStruct((2, _C, _C), jnp.bfloat16),
    )(a)


def _prep_body(x_ref, degn_ref, degf_ref, y1_ref, nn_ref, nf_ref, nx_ref):
    x = x_ref[...]
    nn = lax.rsqrt(jnp.maximum(degn_ref[...], 1.0))
    nf = lax.rsqrt(jnp.maximum(degf_ref[...], 1.0))
    nn_ref[...] = nn
    nf_ref[...] = nf
    y1_ref[...] = (x * nn).astype(jnp.bfloat16)
    x0 = x[:, :_D]
    nx_ref[...] = jnp.sum(x0 * x0, axis=1, keepdims=True)


def _prep(x, degn, degf):
    return pl.pallas_call(
        _prep_body,
        grid=(_MB,),
        in_specs=[
            pl.BlockSpec((_BM, _W), lambda i: (i, 0)),
            pl.BlockSpec((_BM, 1), lambda i: (i, 0)),
            pl.BlockSpec((_BM, 1), lambda i: (i, 0)),
        ],
        out_specs=[
            pl.BlockSpec((_BM, _W), lambda i: (i, 0)),
            pl.BlockSpec((_BM, 1), lambda i: (i, 0)),
            pl.BlockSpec((_BM, 1), lambda i: (i, 0)),
            pl.BlockSpec((_BM, 1), lambda i: (i, 0)),
        ],
        out_shape=[
            jax.ShapeDtypeStruct((_C, _W), jnp.bfloat16),
            jax.ShapeDtypeStruct((_C, 1), jnp.float32),
            jax.ShapeDtypeStruct((_C, 1), jnp.float32),
            jax.ShapeDtypeStruct((_C, 1), jnp.float32),
        ],
    )(x, degn, degf)


def _layer_body(relu, final, a_ref, y_ref, no_ref, nn_ref, w_ref, b_ref,
                o_ref):
    a = a_ref[...]
    y = y_ref[...]
    acc = jnp.dot(a, y, preferred_element_type=jnp.float32)
    z = (acc * no_ref[...]).astype(jnp.bfloat16)
    wmat = w_ref[...].astype(jnp.bfloat16)
    b = b_ref[...]
    cols = []
    for j in range(_NBATCH):
        h = jnp.dot(z[:, j * _D:(j + 1) * _D], wmat,
                    preferred_element_type=jnp.float32) + b
        if relu:
            h = jnp.maximum(h, 0.0)
        cols.append(h)
    h = jnp.concatenate(cols, axis=1)
    if final:
        o_ref[...] = h
    else:
        o_ref[...] = (h * nn_ref[...]).astype(jnp.bfloat16)


def _layer(a, y, norm_out, norm_next, wmat, b, relu, final):
    odtype = jnp.float32 if final else jnp.bfloat16
    return pl.pallas_call(
        functools.partial(_layer_body, relu, final),
        grid=(_MB,),
        in_specs=[
            pl.BlockSpec((_BM, _C), lambda i: (i, 0)),
            pl.BlockSpec((_C, _W), lambda i: (0, 0)),
            pl.BlockSpec((_BM, 1), lambda i: (i, 0)),
            pl.BlockSpec((_BM, 1), lambda i: (i, 0)),
            pl.BlockSpec((_D, _D), lambda i: (0, 0)),
            pl.BlockSpec((1, _D), lambda i: (0, 0)),
        ],
        out_specs=pl.BlockSpec((_BM, _W), lambda i: (i, 0)),
        out_shape=jax.ShapeDtypeStruct((_C, _W), odtype),
    )(a, y, norm_out, norm_next, wmat, b.reshape(1, _D))


_BG = 200


def _gram_body(xa_ref, xb_ref, g_ref):
    xa = xa_ref[...].astype(jnp.bfloat16)
    xb = xb_ref[...].astype(jnp.bfloat16)
    g_ref[...] = lax.dot_general(xa, xb, (((1,), (1,)), ((), ())),
                                 preferred_element_type=jnp.float32)


def _gram(x0):
    return pl.pallas_call(
        _gram_body,
        grid=(_C // _BG,),
        in_specs=[
            pl.BlockSpec((_BG, _D), lambda i: (i, 0)),
            pl.BlockSpec((_C, _D), lambda i: (0, 0)),
        ],
        out_specs=pl.BlockSpec((_BG, _C), lambda i: (i, 0)),
        out_shape=jax.ShapeDtypeStruct((_C, _C), jnp.float32),
    )(x0, x0)


_ER, _EC = 1250, 128


def _loss_body(vn_ref, vf_ref, uf_ref, vfi_ref, al_ref, mg_ref, o_ref):
    margin = mg_ref[0, 0]
    gn = vn_ref[0]
    dt2n = jnp.maximum(vn_ref[1] + vn_ref[2] - 2.0 * gn, 0.0)
    touches = jnp.sum(dt2n) / _E

    eq = uf_ref[...] == vfi_ref[...]
    dt2f = jnp.where(eq, 0.0, jnp.maximum(vf_ref[1] + vf_ref[2]
                                          - 2.0 * vf_ref[0], 0.0))
    dd = jnp.sqrt(dt2f)
    hinge = jnp.maximum(margin - dd, 0.0)
    disjoint = jnp.sum(hinge * hinge) / _E

    a = jax.nn.sigmoid(al_ref[0, 0])
    o_ref[...] = jnp.reshape(a * touches + (1.0 - a) * disjoint, (1, 1))


def _loss(vn, vf, uf, vfi, alpha, margin):
    return pl.pallas_call(
        _loss_body,
        grid=(1,),
        in_specs=[
            pl.BlockSpec((3, _ER, _EC), lambda i: (0, 0, 0)),
            pl.BlockSpec((3, _ER, _EC), lambda i: (0, 0, 0)),
            pl.BlockSpec((_ER, _EC), lambda i: (0, 0)),
            pl.BlockSpec((_ER, _EC), lambda i: (0, 0)),
            pl.BlockSpec((1, 1), lambda i: (0, 0)),
            pl.BlockSpec((1, 1), lambda i: (0, 0)),
        ],
        out_specs=pl.BlockSpec((1, 1), lambda i: (0, 0)),
        out_shape=jax.ShapeDtypeStruct((1, 1), jnp.float32),
    )(vn, vf, uf, vfi, alpha, margin)


def kernel(node_feats, u_near, v_near, u_far, v_far,
           W1n, b1n, W2n, b2n, W1f, b1f, W2f, b2f, alpha, margin):
    edges = jnp.concatenate([u_near, v_near, u_far, v_far])

    adj, deg = _build_adj(edges)
    adj = _cast_bf16(adj.reshape(2, _C, _C))
    deg = deg.reshape(2, _C)

    x = node_feats.transpose(1, 0, 2).reshape(_C, _W)
    y1, nn, nf, nx = _prep(x, deg[0].reshape(_C, 1), deg[1].reshape(_C, 1))

    y2 = _layer(adj[0], y1, nn, nn, W1n, b1n, relu=True, final=False)
    y3 = _layer(adj[0], y2, nn, nf, W2n, b2n, relu=False, final=False)
    y4 = _layer(adj[1], y3, nf, nf, W1f, b1f, relu=True, final=False)
    far = _layer(adj[1], y4, nf, nf, W2f, b2f, relu=False, final=True)
    new_node_feats = far.reshape(_C, _NBATCH, _D).transpose(1, 0, 2)

    gmat = _gram(node_feats[0])
    vals = _loss_gather(edges, gmat.reshape(-1), nx.reshape(-1))
    vals = vals.reshape(2, 3, _E)
    vn = vals[0].reshape(3, _ER, _EC)
    vf = vals[1].reshape(3, _ER, _EC)
    topo = _loss(vn, vf, u_far.reshape(_ER, _EC), v_far.reshape(_ER, _EC),
                 alpha.reshape(1, 1), margin.reshape(1, 1))[0, 0]

    return new_node_feats, topo


# speedup vs baseline: 40.5719x; 1.0658x over previous
import functools

import jax
import jax.numpy as jnp
from jax import lax
from jax.experimental import pallas as pl
from jax.experimental.pallas import tpu as pltpu
from jax.experimental.pallas import tpu_sc as plsc

_C = 10000
_E = 160000
_NBATCH = 4
_D = 256
_W = _NBATCH * _D
_NS = 16
_EP = _E // _NS
_WROWS = 100
_NWIN = _C // _WROWS
_WIN_TOT = _WROWS + 2
_CHUNK = 2000
_NCHUNK = _EP // _CHUNK
_VEC = 16
_BM = 400
_MB = _C // _BM


def _fill(ref, n, vec):
    def body(i, _):
        ref[pl.ds(i * _VEC, _VEC)] = vec
        return 0
    lax.fori_loop(0, n // _VEC, body, 0)


def _adj_body(edges, a_out, ubuf, vbuf, idx0, idx1, ones_ck, zbuf,
              stage0, sems, win_sh):
    g = lax.axis_index("c")
    s = lax.axis_index("s")

    _fill(zbuf, _C, jnp.zeros((_VEC,), jnp.float32))
    _fill(ones_ck, 2 * _CHUNK, jnp.ones((_VEC,), jnp.float32))

    pltpu.sync_copy(edges.at[pl.ds(g * 2 * _E + s * _EP, _EP)], ubuf)
    pltpu.sync_copy(edges.at[pl.ds(g * 2 * _E + _E + s * _EP, _EP)], vbuf)

    idxbufs = (idx0, idx1)

    def zero_rows(i, _):
        r = s + i * _NS

        @pl.when(r < _WROWS)
        def _():
            pltpu.sync_copy(zbuf, win_sh.at[pl.ds((r + 1) * _C, _C)])
        return 0
    lax.fori_loop(0, (_WROWS + _NS - 1) // _NS, zero_rows, 0)
    plsc.subcore_barrier()

    def win_body(w, _):
        base = w * _WROWS

        def compute_chunk(c, buf):
            def scan(i, _):
                off = c * _CHUNK + i * _VEC
                uv = ubuf[pl.ds(off, _VEC)]
                vv = vbuf[pl.ds(off, _VEC)]
                relv = jnp.clip(vv - base, -1, _WROWS)
                buf[pl.ds(i * _VEC, _VEC)] = (relv + 1) * _C + uv
                relu = jnp.clip(uv - base, -1, _WROWS)
                buf[pl.ds(_CHUNK + i * _VEC, _VEC)] = (relu + 1) * _C + vv
                return 0
            lax.fori_loop(0, _CHUNK // _VEC, scan, 0)

        compute_chunk(0, idx0)
        for c in range(_NCHUNK):
            d = pltpu.async_copy(ones_ck, win_sh.at[idxbufs[c % 2]],
                                 sems[1 + c % 2], add=True)
            if c + 1 < _NCHUNK:
                compute_chunk(c + 1, idxbufs[(c + 1) % 2])
            d.wait()
        plsc.subcore_barrier()

        def writeback(i, _):
            r = s + i * _NS

            @pl.when(r < _WROWS)
            def _():
                pltpu.sync_copy(win_sh.at[pl.ds((r + 1) * _C, _C)], stage0)
                d2 = pltpu.async_copy(
                    stage0, a_out.at[pl.ds((g * _C + base + r) * _C, _C)],
                    sems[3])
                pltpu.sync_copy(zbuf, win_sh.at[pl.ds((r + 1) * _C, _C)])
                d2.wait()
            return 0
        lax.fori_loop(0, (_WROWS + _NS - 1) // _NS, writeback, 0)
        plsc.subcore_barrier()
        return 0

    lax.fori_loop(0, _NWIN, win_body, 0)


def _build_adj(edges):
    mesh = plsc.VectorSubcoreMesh(core_axis_name="c", subcore_axis_name="s")
    f = pl.kernel(
        _adj_body,
        out_type=jax.ShapeDtypeStruct((2 * _C * _C,), jnp.float32),
        mesh=mesh,
        scratch_types=[
            pltpu.VMEM((_EP,), jnp.int32),
            pltpu.VMEM((_EP,), jnp.int32),
            pltpu.VMEM((2 * _CHUNK,), jnp.int32),
            pltpu.VMEM((2 * _CHUNK,), jnp.int32),
            pltpu.VMEM((2 * _CHUNK,), jnp.float32),
            pltpu.VMEM((_C,), jnp.float32),
            pltpu.VMEM((_C,), jnp.float32),
            [pltpu.SemaphoreType.DMA] * 4,
            pltpu.VMEM_SHARED((_WIN_TOT * _C,), jnp.float32),
        ],
    )
    return f(edges)


def _deg_body(edges, deg_out, ubuf, vbuf, zbuf, ones_ep, stagebuf, deg_sh):
    g = lax.axis_index("c")
    s = lax.axis_index("s")
    _fill(zbuf, _C, jnp.zeros((_VEC,), jnp.float32))
    _fill(ones_ep, _EP, jnp.ones((_VEC,), jnp.float32))
    pltpu.sync_copy(edges.at[pl.ds(g * 2 * _E + s * _EP, _EP)], ubuf)
    pltpu.sync_copy(edges.at[pl.ds(g * 2 * _E + _E + s * _EP, _EP)], vbuf)

    @pl.when(s == 0)
    def _():
        pltpu.sync_copy(zbuf, deg_sh)
    plsc.subcore_barrier()
    pltpu.sync_copy(ones_ep, deg_sh.at[ubuf], add=True)
    pltpu.sync_copy(ones_ep, deg_sh.at[vbuf], add=True)
    plsc.subcore_barrier()

    @pl.when(s == 0)
    def _():
        pltpu.sync_copy(deg_sh, stagebuf)
        pltpu.sync_copy(stagebuf, deg_out.at[pl.ds(g * _C, _C)])


def _build_deg(edges):
    mesh = plsc.VectorSubcoreMesh(core_axis_name="c", subcore_axis_name="s")
    f = pl.kernel(
        _deg_body,
        out_type=jax.ShapeDtypeStruct((2 * _C,), jnp.float32),
        mesh=mesh,
        scratch_types=[
            pltpu.VMEM((_EP,), jnp.int32),
            pltpu.VMEM((_EP,), jnp.int32),
            pltpu.VMEM((_C,), jnp.float32),
            pltpu.VMEM((_EP,), jnp.float32),
            pltpu.VMEM((_C,), jnp.float32),
            pltpu.VMEM_SHARED((_C,), jnp.float32),
        ],
    )
    return f(edges)


def _loss_gather_body(edges, gmat, nx, out, ubuf, vbuf, idxbuf, valbuf):
    g = lax.axis_index("c")
    s = lax.axis_index("s")
    pltpu.sync_copy(edges.at[pl.ds(g * 2 * _E + s * _EP, _EP)], ubuf)
    pltpu.sync_copy(edges.at[pl.ds(g * 2 * _E + _E + s * _EP, _EP)], vbuf)

    def body(i, _):
        uv = ubuf[pl.ds(i * _VEC, _VEC)]
        vv = vbuf[pl.ds(i * _VEC, _VEC)]
        idxbuf[pl.ds(i * _VEC, _VEC)] = uv * _C + vv
        return 0
    lax.fori_loop(0, _EP // _VEC, body, 0)

    base = g * 3 * _E + s * _EP
    pltpu.sync_copy(gmat.at[idxbuf], valbuf)
    pltpu.sync_copy(valbuf, out.at[pl.ds(base, _EP)])
    pltpu.sync_copy(nx.at[ubuf], valbuf)
    pltpu.sync_copy(valbuf, out.at[pl.ds(base + _E, _EP)])
    pltpu.sync_copy(nx.at[vbuf], valbuf)
    pltpu.sync_copy(valbuf, out.at[pl.ds(base + 2 * _E, _EP)])


def _loss_gather(edges, gmat_flat, nx_flat):
    mesh = plsc.VectorSubcoreMesh(core_axis_name="c", subcore_axis_name="s")
    f = pl.kernel(
        _loss_gather_body,
        out_type=jax.ShapeDtypeStruct((2 * 3 * _E,), jnp.float32),
        mesh=mesh,
        scratch_types=[
            pltpu.VMEM((_EP,), jnp.int32),
            pltpu.VMEM((_EP,), jnp.int32),
            pltpu.VMEM((_EP,), jnp.int32),
            pltpu.VMEM((_EP,), jnp.float32),
        ],
    )
    return f(edges, gmat_flat, nx_flat)


def _cast_body(a_ref, o_ref):
    o_ref[...] = a_ref[...].astype(jnp.bfloat16)


def _cast_bf16(a):
    return pl.pallas_call(
        _cast_body,
        grid=(2, _MB),
        in_specs=[pl.BlockSpec((1, _BM, _C), lambda g, i: (g, i, 0))],
        out_specs=pl.BlockSpec((1, _BM, _C), lambda g, i: (g, i, 0)),
        out_shape=jax.ShapeDtypeStruct((2, _C, _C), jnp.bfloat16),
    )(a)


def _prep_body(x_ref, degn_ref, degf_ref, y1_ref, nn_ref, nf_ref, nx_ref):
    x = x_ref[...]
    nn = lax.rsqrt(jnp.maximum(degn_ref[...], 1.0))
    nf = lax.rsqrt(jnp.maximum(degf_ref[...], 1.0))
    nn_ref[...] = nn
    nf_ref[...] = nf
    y1_ref[...] = (x * nn).astype(jnp.bfloat16)
    x0 = x[:, :_D]
    nx_ref[...] = jnp.sum(x0 * x0, axis=1, keepdims=True)


def _prep(x, degn, degf):
    return pl.pallas_call(
        _prep_body,
        grid=(_MB,),
        in_specs=[
            pl.BlockSpec((_BM, _W), lambda i: (i, 0)),
            pl.BlockSpec((_BM, 1), lambda i: (i, 0)),
            pl.BlockSpec((_BM, 1), lambda i: (i, 0)),
        ],
        out_specs=[
            pl.BlockSpec((_BM, _W), lambda i: (i, 0)),
            pl.BlockSpec((_BM, 1), lambda i: (i, 0)),
            pl.BlockSpec((_BM, 1), lambda i: (i, 0)),
            pl.BlockSpec((_BM, 1), lambda i: (i, 0)),
        ],
        out_shape=[
            jax.ShapeDtypeStruct((_C, _W), jnp.bfloat16),
            jax.ShapeDtypeStruct((_C, 1), jnp.float32),
            jax.ShapeDtypeStruct((_C, 1), jnp.float32),
            jax.ShapeDtypeStruct((_C, 1), jnp.float32),
        ],
    )(x, degn, degf)


def _layer_body(relu, final, a_ref, y_ref, no_ref, nn_ref, w_ref, b_ref,
                o_ref):
    a = a_ref[...]
    y = y_ref[...]
    acc = jnp.dot(a, y, preferred_element_type=jnp.float32)
    z = (acc * no_ref[...]).astype(jnp.bfloat16)
    wmat = w_ref[...].astype(jnp.bfloat16)
    b = b_ref[...]
    cols = []
    for j in range(_NBATCH):
        h = jnp.dot(z[:, j * _D:(j + 1) * _D], wmat,
                    preferred_element_type=jnp.float32) + b
        if relu:
            h = jnp.maximum(h, 0.0)
        cols.append(h)
    h = jnp.concatenate(cols, axis=1)
    if final:
        o_ref[...] = h
    else:
        o_ref[...] = (h * nn_ref[...]).astype(jnp.bfloat16)


def _layer(a, y, norm_out, norm_next, wmat, b, relu, final):
    odtype = jnp.float32 if final else jnp.bfloat16
    return pl.pallas_call(
        functools.partial(_layer_body, relu, final),
        grid=(_MB,),
        in_specs=[
            pl.BlockSpec((_BM, _C), lambda i: (i, 0)),
            pl.BlockSpec((_C, _W), lambda i: (0, 0)),
            pl.BlockSpec((_BM, 1), lambda i: (i, 0)),
            pl.BlockSpec((_BM, 1), lambda i: (i, 0)),
            pl.BlockSpec((_D, _D), lambda i: (0, 0)),
            pl.BlockSpec((1, _D), lambda i: (0, 0)),
        ],
        out_specs=pl.BlockSpec((_BM, _W), lambda i: (i, 0)),
        out_shape=jax.ShapeDtypeStruct((_C, _W), odtype),
    )(a, y, norm_out, norm_next, wmat, b.reshape(1, _D))


_BG = 200


def _gram_body(xa_ref, xb_ref, g_ref):
    xa = xa_ref[...].astype(jnp.bfloat16)
    xb = xb_ref[...].astype(jnp.bfloat16)
    g_ref[...] = lax.dot_general(xa, xb, (((1,), (1,)), ((), ())),
                                 preferred_element_type=jnp.float32)


def _gram(x0):
    return pl.pallas_call(
        _gram_body,
        grid=(_C // _BG,),
        in_specs=[
            pl.BlockSpec((_BG, _D), lambda i: (i, 0)),
            pl.BlockSpec((_C, _D), lambda i: (0, 0)),
        ],
        out_specs=pl.BlockSpec((_BG, _C), lambda i: (i, 0)),
        out_shape=jax.ShapeDtypeStruct((_C, _C), jnp.float32),
    )(x0, x0)


_ER, _EC = 1250, 128


def _loss_body(vn_ref, vf_ref, uf_ref, vfi_ref, al_ref, mg_ref, o_ref):
    margin = mg_ref[0, 0]
    gn = vn_ref[0]
    dt2n = jnp.maximum(vn_ref[1] + vn_ref[2] - 2.0 * gn, 0.0)
    touches = jnp.sum(dt2n) / _E

    eq = uf_ref[...] == vfi_ref[...]
    dt2f = jnp.where(eq, 0.0, jnp.maximum(vf_ref[1] + vf_ref[2]
                                          - 2.0 * vf_ref[0], 0.0))
    dd = jnp.sqrt(dt2f)
    hinge = jnp.maximum(margin - dd, 0.0)
    disjoint = jnp.sum(hinge * hinge) / _E

    a = jax.nn.sigmoid(al_ref[0, 0])
    o_ref[...] = jnp.reshape(a * touches + (1.0 - a) * disjoint, (1, 1))


def _loss(vn, vf, uf, vfi, alpha, margin):
    return pl.pallas_call(
        _loss_body,
        grid=(1,),
        in_specs=[
            pl.BlockSpec((3, _ER, _EC), lambda i: (0, 0, 0)),
            pl.BlockSpec((3, _ER, _EC), lambda i: (0, 0, 0)),
            pl.BlockSpec((_ER, _EC), lambda i: (0, 0)),
            pl.BlockSpec((_ER, _EC), lambda i: (0, 0)),
            pl.BlockSpec((1, 1), lambda i: (0, 0)),
            pl.BlockSpec((1, 1), lambda i: (0, 0)),
        ],
        out_specs=pl.BlockSpec((1, 1), lambda i: (0, 0)),
        out_shape=jax.ShapeDtypeStruct((1, 1), jnp.float32),
    )(vn, vf, uf, vfi, alpha, margin)


def kernel(node_feats, u_near, v_near, u_far, v_far,
           W1n, b1n, W2n, b2n, W1f, b1f, W2f, b2f, alpha, margin):
    edges = jnp.concatenate([u_near, v_near, u_far, v_far])

    deg = _build_deg(edges).reshape(2, _C)
    adj = _build_adj(edges)
    adj = _cast_bf16(adj.reshape(2, _C, _C))

    x = node_feats.transpose(1, 0, 2).reshape(_C, _W)
    y1, nn, nf, nx = _prep(x, deg[0].reshape(_C, 1), deg[1].reshape(_C, 1))
    gmat = _gram(node_feats[0])
    vals = _loss_gather(edges, gmat.reshape(-1), nx.reshape(-1))

    y2 = _layer(adj[0], y1, nn, nn, W1n, b1n, relu=True, final=False)
    y3 = _layer(adj[0], y2, nn, nf, W2n, b2n, relu=False, final=False)
    y4 = _layer(adj[1], y3, nf, nf, W1f, b1f, relu=True, final=False)
    far = _layer(adj[1], y4, nf, nf, W2f, b2f, relu=False, final=True)
    new_node_feats = far.reshape(_C, _NBATCH, _D).transpose(1, 0, 2)

    vals = vals.reshape(2, 3, _E)
    vn = vals[0].reshape(3, _ER, _EC)
    vf = vals[1].reshape(3, _ER, _EC)
    topo = _loss(vn, vf, u_far.reshape(_ER, _EC), v_far.reshape(_ER, _EC),
                 alpha.reshape(1, 1), margin.reshape(1, 1))[0, 0]

    return new_node_feats, topo
